# baseline (speedup 1.0000x reference)
import functools

import jax
import jax.numpy as jnp
from jax import lax
from jax.experimental import pallas as pl
from jax.experimental.pallas import tpu as pltpu
from jax.experimental.pallas import tpu_sc as plsc

D = 64
MAX_LEN = 5000
B, T = 4096, 200
ROWS = B * T
ROWS2 = ROWS // 2
NC, NS = 2, 16
NW = NC * NS
RPW = ROWS // NW
CHUNK = 512
CHUNK2 = CHUNK // 2
NCHUNK = RPW // CHUNK
IDXB = 128
NGATHER = CHUNK // IDXB

_mesh = plsc.VectorSubcoreMesh(core_axis_name="c", subcore_axis_name="s")


@functools.partial(
    pl.kernel,
    out_type=jax.ShapeDtypeStruct((ROWS2, 2 * D), jnp.float32),
    mesh=_mesh,
    scratch_types=[
        pltpu.VMEM((CHUNK,), jnp.float32),
        pltpu.VMEM((NGATHER, IDXB), jnp.int32),
        pltpu.VMEM((CHUNK2, 2 * D), jnp.float32),
        pltpu.VMEM((CHUNK, D), jnp.float32),
        pltpu.SemaphoreType.DMA,
        pltpu.SemaphoreType.DMA,
    ],
    compiler_params=pltpu.CompilerParams(use_tc_tiling_on_sc=False),
)
def _sc_add_pe(x_hbm, tf_hbm, pe_hbm, out_hbm, tf_v, idx_v, x_v, pe_v,
               sem_x, sem_g):
    wid = lax.axis_index("s") * NC + lax.axis_index("c")
    base = wid * RPW

    def chunk_body(c, carry):
        row0 = base + c * CHUNK
        cp_x = pltpu.async_copy(
            x_hbm.at[pl.ds(row0 // 2, CHUNK2)], x_v, sem_x)
        pltpu.sync_copy(tf_hbm.at[pl.ds(row0, CHUNK)], tf_v)

        def idx_body(i, _):
            t = tf_v[pl.ds(i * 16, 16)]
            iv = (t * float(MAX_LEN)).astype(jnp.int32)
            idx_v[i // (IDXB // 16), pl.ds((i % (IDXB // 16)) * 16, 16)] = iv
            return 0

        lax.fori_loop(0, CHUNK // 16, idx_body, 0)

        gathers = []
        for j in range(NGATHER):
            gathers.append(pltpu.async_copy(
                pe_hbm.at[idx_v.at[j]], pe_v.at[pl.ds(j * IDXB, IDXB)], sem_g))
        cp_x.wait()
        for g in gathers:
            g.wait()

        def add_body(j, _):
            for h in range(2 * D // 16):
                s = pl.ds(h * 16, 16)
                sp = pl.ds((h % 4) * 16, 16)
                plsc.addupdate(x_v.at[j, s], pe_v[2 * j + h // 4, sp])
            return 0

        lax.fori_loop(0, CHUNK2, add_body, 0)
        pltpu.sync_copy(x_v, out_hbm.at[pl.ds(row0 // 2, CHUNK2)])
        return carry

    lax.fori_loop(0, NCHUNK, chunk_body, 0)


def kernel(x, time_features, pe):
    out = _sc_add_pe(x.reshape(ROWS2, 2 * D), time_features.reshape(ROWS), pe)
    return out.reshape(B, T, D)


# speedup vs baseline: 1.4070x; 1.4070x over previous
import functools

import jax
import jax.numpy as jnp
from jax import lax
from jax.experimental import pallas as pl
from jax.experimental.pallas import tpu as pltpu
from jax.experimental.pallas import tpu_sc as plsc

D = 64
MAX_LEN = 5000
B, T = 4096, 200
ROWS = B * T
NC, NS = 2, 16
NW = NC * NS
RPW = ROWS // NW
CHUNK = 256
NCHUNK = RPW // CHUNK
IDXB = 128
NGATHER = CHUNK // IDXB

_mesh = plsc.VectorSubcoreMesh(core_axis_name="c", subcore_axis_name="s")


@functools.partial(
    pl.kernel,
    out_type=jax.ShapeDtypeStruct((ROWS, D), jnp.float32),
    mesh=_mesh,
    scratch_types=[
        pltpu.VMEM((CHUNK,), jnp.float32),
        pltpu.VMEM((NGATHER, IDXB), jnp.int32),
        pltpu.VMEM((CHUNK, D), jnp.float32),
        pltpu.VMEM((CHUNK, 2 * D), jnp.float32),
        pltpu.SemaphoreType.DMA,
        pltpu.SemaphoreType.DMA,
    ],
)
def _sc_add_pe(x_hbm, tf_hbm, pe_hbm, out_hbm, tf_v, idx_v, x_v, pe_v,
               sem_x, sem_g):
    wid = lax.axis_index("s") * NC + lax.axis_index("c")
    base = wid * RPW

    def chunk_body(c, carry):
        row0 = base + c * CHUNK
        cp_x = pltpu.async_copy(x_hbm.at[pl.ds(row0, CHUNK)], x_v, sem_x)
        pltpu.sync_copy(tf_hbm.at[pl.ds(row0, CHUNK)], tf_v)

        def idx_body(i, _):
            t = tf_v[pl.ds(i * 16, 16)]
            iv = (t * float(MAX_LEN)).astype(jnp.int32)
            idx_v[i // (IDXB // 16), pl.ds((i % (IDXB // 16)) * 16, 16)] = iv
            return 0

        lax.fori_loop(0, CHUNK // 16, idx_body, 0)

        gathers = []
        for j in range(NGATHER):
            gathers.append(pltpu.async_copy(
                pe_hbm.at[idx_v.at[j]], pe_v.at[pl.ds(j * IDXB, IDXB)], sem_g))
        cp_x.wait()
        for g in gathers:
            g.wait()

        def add_body(j, _):
            for h in range(D // 16):
                s = pl.ds(h * 16, 16)
                plsc.addupdate(x_v.at[j, s], pe_v[j, s])
            return 0

        lax.fori_loop(0, CHUNK, add_body, 0)
        pltpu.sync_copy(x_v, out_hbm.at[pl.ds(row0, CHUNK)])
        return carry

    lax.fori_loop(0, NCHUNK, chunk_body, 0)


def kernel(x, time_features, pe):
    pe2 = jnp.concatenate([pe, pe], axis=1)
    out = _sc_add_pe(x.reshape(ROWS, D), time_features.reshape(ROWS), pe2)
    return out.reshape(B, T, D)


# speedup vs baseline: 1.5582x; 1.1075x over previous
import functools

import jax
import jax.numpy as jnp
from jax import lax
from jax.experimental import pallas as pl
from jax.experimental.pallas import tpu as pltpu
from jax.experimental.pallas import tpu_sc as plsc

D = 64
MAX_LEN = 5000
B, T = 4096, 200
ROWS = B * T
NC, NS = 2, 16
NW = NC * NS
RPW = ROWS // NW
CHUNK = 256
IDXB = 128
NGATHER = CHUNK // IDXB
NITER = RPW // (2 * CHUNK)

_mesh = plsc.VectorSubcoreMesh(core_axis_name="c", subcore_axis_name="s")


@functools.partial(
    pl.kernel,
    out_type=jax.ShapeDtypeStruct((ROWS, D), jnp.float32),
    mesh=_mesh,
    scratch_types=[
        pltpu.VMEM((2, CHUNK), jnp.float32),
        pltpu.VMEM((2, NGATHER, IDXB), jnp.int32),
        pltpu.VMEM((2, CHUNK, D), jnp.float32),
        pltpu.VMEM((CHUNK, 2 * D), jnp.float32),
        pltpu.SemaphoreType.DMA,
        pltpu.SemaphoreType.DMA,
        pltpu.SemaphoreType.DMA,
        pltpu.SemaphoreType.DMA,
        pltpu.SemaphoreType.DMA,
        pltpu.SemaphoreType.DMA,
        pltpu.SemaphoreType.DMA,
        pltpu.SemaphoreType.DMA,
    ],
)
def _sc_add_pe(x_hbm, tf_hbm, pe_hbm, out_hbm, tf_v, idx_v, x_v, pe_v,
               s_tf0, s_tf1, s_x0, s_x1, s_g0, s_g1, s_o0, s_o1):
    wid = lax.axis_index("s") * NC + lax.axis_index("c")
    base = wid * RPW
    s_tf = (s_tf0, s_tf1)
    s_x = (s_x0, s_x1)
    s_g = (s_g0, s_g1)
    s_o = (s_o0, s_o1)

    def start_in(slot, row0):
        pltpu.async_copy(tf_hbm.at[pl.ds(row0, CHUNK)], tf_v.at[slot],
                         s_tf[slot])
        pltpu.async_copy(x_hbm.at[pl.ds(row0, CHUNK)], x_v.at[slot],
                         s_x[slot])

    def wait_in(slot, row0):
        pltpu.make_async_copy(tf_hbm.at[pl.ds(row0, CHUNK)], tf_v.at[slot],
                              s_tf[slot]).wait()

    def compute_idx_and_gather(slot):
        def idx_body(i, _):
            t = tf_v[slot, pl.ds(i * 16, 16)]
            iv = (t * float(MAX_LEN)).astype(jnp.int32)
            idx_v[slot, i // (IDXB // 16),
                  pl.ds((i % (IDXB // 16)) * 16, 16)] = iv
            return 0

        lax.fori_loop(0, CHUNK // 16, idx_body, 0)
        for j in range(NGATHER):
            pltpu.async_copy(pe_hbm.at[idx_v.at[slot, j]],
                             pe_v.at[pl.ds(j * IDXB, IDXB)], s_g[slot])

    def wait_and_add(slot, row0):
        pltpu.make_async_copy(x_hbm.at[pl.ds(row0, CHUNK)], x_v.at[slot],
                              s_x[slot]).wait()
        for j in range(NGATHER):
            pltpu.make_async_copy(pe_hbm.at[idx_v.at[slot, j]],
                                  pe_v.at[pl.ds(j * IDXB, IDXB)],
                                  s_g[slot]).wait()

        def add_body(j, _):
            for h in range(D // 16):
                s = pl.ds(h * 16, 16)
                plsc.addupdate(x_v.at[slot, j, s], pe_v[j, s])
            return 0

        lax.fori_loop(0, CHUNK, add_body, 0)

    def start_out(slot, row0):
        pltpu.async_copy(x_v.at[slot], out_hbm.at[pl.ds(row0, CHUNK)],
                         s_o[slot])

    def wait_out(slot, row0):
        pltpu.make_async_copy(x_v.at[slot], out_hbm.at[pl.ds(row0, CHUNK)],
                              s_o[slot]).wait()

    start_in(0, base)

    def pair_body(k, carry):
        row_a = base + (2 * k) * CHUNK
        row_b = row_a + CHUNK
        row_a2 = jnp.minimum(row_a + 2 * CHUNK, ROWS - CHUNK)

        start_in(1, row_b)
        wait_in(0, row_a)
        compute_idx_and_gather(0)
        wait_and_add(0, row_a)
        start_out(0, row_a)
        wait_in(1, row_b)
        compute_idx_and_gather(1)
        wait_out(0, row_a)
        start_in(0, row_a2)
        wait_and_add(1, row_b)
        start_out(1, row_b)
        wait_out(1, row_b)
        return carry

    lax.fori_loop(0, NITER, pair_body, 0)
    wait_in(0, ROWS - CHUNK)
    pltpu.make_async_copy(x_hbm.at[pl.ds(ROWS - CHUNK, CHUNK)], x_v.at[0],
                          s_x0).wait()


def kernel(x, time_features, pe):
    pe2 = jnp.concatenate([pe, pe], axis=1)
    out = _sc_add_pe(x.reshape(ROWS, D), time_features.reshape(ROWS), pe2)
    return out.reshape(B, T, D)


# speedup vs baseline: 1.5654x; 1.0046x over previous
import functools

import jax
import jax.numpy as jnp
from jax import lax
from jax.experimental import pallas as pl
from jax.experimental.pallas import tpu as pltpu
from jax.experimental.pallas import tpu_sc as plsc

D = 64
MAX_LEN = 5000
B, T = 4096, 200
ROWS = B * T
NC, NS = 2, 16
NW = NC * NS
RPW = ROWS // NW
CHUNK = 256
IDXB = 128
NGATHER = CHUNK // IDXB
NITER = RPW // (2 * CHUNK)

_mesh = plsc.VectorSubcoreMesh(core_axis_name="c", subcore_axis_name="s")


@functools.partial(
    pl.kernel,
    out_type=jax.ShapeDtypeStruct((ROWS, D), jnp.float32),
    mesh=_mesh,
    scratch_types=[
        pltpu.VMEM((2, CHUNK), jnp.float32),
        pltpu.VMEM((2, NGATHER, IDXB), jnp.int32),
        pltpu.VMEM((2, CHUNK, D), jnp.float32),
        pltpu.VMEM((CHUNK, 2 * D), jnp.float32),
        pltpu.SemaphoreType.DMA,
        pltpu.SemaphoreType.DMA,
        pltpu.SemaphoreType.DMA,
        pltpu.SemaphoreType.DMA,
        pltpu.SemaphoreType.DMA,
        pltpu.SemaphoreType.DMA,
        pltpu.SemaphoreType.DMA,
        pltpu.SemaphoreType.DMA,
    ],
)
def _sc_add_pe(x_hbm, tf_hbm, pe_hbm, out_hbm, tf_v, idx_v, x_v, pe_v,
               s_tf0, s_tf1, s_x0, s_x1, s_g0, s_g1, s_o0, s_o1):
    wid = lax.axis_index("s") * NC + lax.axis_index("c")
    base = wid * RPW
    s_tf = (s_tf0, s_tf1)
    s_x = (s_x0, s_x1)
    s_g = (s_g0, s_g1)
    s_o = (s_o0, s_o1)

    def start_in(slot, row0):
        pltpu.async_copy(tf_hbm.at[pl.ds(row0, CHUNK)], tf_v.at[slot],
                         s_tf[slot])
        pltpu.async_copy(x_hbm.at[pl.ds(row0, CHUNK)], x_v.at[slot],
                         s_x[slot])

    def wait_in(slot, row0):
        pltpu.make_async_copy(tf_hbm.at[pl.ds(row0, CHUNK)], tf_v.at[slot],
                              s_tf[slot]).wait()

    def compute_idx_and_gather(slot):
        def idx_body(i, _):
            t = tf_v[slot, pl.ds(i * 16, 16)]
            iv = (t * float(MAX_LEN)).astype(jnp.int32)
            idx_v[slot, i // (IDXB // 16),
                  pl.ds((i % (IDXB // 16)) * 16, 16)] = iv
            return 0

        lax.fori_loop(0, CHUNK // 16, idx_body, 0)
        for j in range(NGATHER):
            pltpu.async_copy(pe_hbm.at[idx_v.at[slot, j]],
                             pe_v.at[pl.ds(j * IDXB, IDXB)], s_g[slot])

    def wait_and_add(slot, row0):
        pltpu.make_async_copy(x_hbm.at[pl.ds(row0, CHUNK)], x_v.at[slot],
                              s_x[slot]).wait()
        for j in range(NGATHER):
            pltpu.make_async_copy(pe_hbm.at[idx_v.at[slot, j]],
                                  pe_v.at[pl.ds(j * IDXB, IDXB)],
                                  s_g[slot]).wait()

        def add_body(j, _):
            for h in range(D // 16):
                s = pl.ds(h * 16, 16)
                plsc.addupdate(x_v.at[slot, j, s], pe_v[j, s])
            return 0

        lax.fori_loop(0, CHUNK, add_body, 0)

    def start_out(slot, row0):
        pltpu.async_copy(x_v.at[slot], out_hbm.at[pl.ds(row0, CHUNK)],
                         s_o[slot])

    def wait_out(slot, row0):
        pltpu.make_async_copy(x_v.at[slot], out_hbm.at[pl.ds(row0, CHUNK)],
                              s_o[slot]).wait()

    start_in(0, base)

    def pair(k, drain_prev_b):
        row_a = base + (2 * k) * CHUNK
        row_b = row_a + CHUNK
        row_a2 = jnp.minimum(row_a + 2 * CHUNK, ROWS - CHUNK)

        if drain_prev_b:
            wait_out(1, row_b - 2 * CHUNK)
        start_in(1, row_b)
        wait_in(0, row_a)
        compute_idx_and_gather(0)
        wait_and_add(0, row_a)
        start_out(0, row_a)
        wait_in(1, row_b)
        compute_idx_and_gather(1)
        wait_out(0, row_a)
        start_in(0, row_a2)
        wait_and_add(1, row_b)
        start_out(1, row_b)

    pair(0, False)

    def pair_body(k, carry):
        pair(k, True)
        return carry

    lax.fori_loop(1, NITER, pair_body, 0)
    wait_out(1, base + RPW - CHUNK)
    wait_in(0, ROWS - CHUNK)
    pltpu.make_async_copy(x_hbm.at[pl.ds(ROWS - CHUNK, CHUNK)], x_v.at[0],
                          s_x0).wait()


def kernel(x, time_features, pe):
    pe2 = jnp.concatenate([pe, pe], axis=1)
    out = _sc_add_pe(x.reshape(ROWS, D), time_features.reshape(ROWS), pe2)
    return out.reshape(B, T, D)
